# baseline (speedup 1.0000x reference)
import functools
import jax
import jax.numpy as jnp
from jax import lax
from jax.experimental import pallas as pl
from jax.experimental.pallas import tpu as pltpu
from jax.experimental.pallas import tpu_sc as plsc

NC = 2
NS = 16
NW = NC * NS
L = 16

_BL = 400


def _cdiv(a, b):
    return -(-a // b)


def _mesh():
    return plsc.VectorSubcoreMesh(core_axis_name="c", subcore_axis_name="s",
                                  num_cores=NC, num_subcores=NS)


def _pad_edges(E):
    blk = min(4096, _cdiv(E, NW * L) * L)
    nb = _cdiv(E, NW * blk)
    return blk, NW * blk * nb


def _pad1(x, n, val):
    if x.shape[0] == n:
        return x
    return jnp.concatenate(
        [x, jnp.full((n - x.shape[0],), val, x.dtype)])


@functools.partial(jax.jit, static_argnums=(2, 3, 4))
def _hist_call(idx_p, val_p, n_out, blk, weighted):
    E_pad = idx_p.shape[0]
    ap = _cdiv(n_out + 1, 128) * 128
    nb = E_pad // (NW * blk)
    share = E_pad // NW

    def body(idx_hbm, val_hbm, out_hbm, accum, idx_s, val_s, sem):
        cid = lax.axis_index("c")
        sid = lax.axis_index("s")
        wid = cid * NS + sid

        def zbody(z, _):
            accum[pl.ds(z * L, L)] = jnp.zeros((L,), jnp.float32)
            return 0
        lax.fori_loop(0, ap // L, zbody, 0)

        for bi in range(nb):
            est = wid * share + bi * blk
            pltpu.sync_copy(idx_hbm.at[pl.ds(est, blk)], idx_s)
            if weighted:
                pltpu.sync_copy(val_hbm.at[pl.ds(est, blk)], val_s)

            def gbody(g, _):
                i16 = idx_s[pl.ds(g * L, L)]
                if weighted:
                    v16 = val_s[pl.ds(g * L, L)]
                else:
                    v16 = jnp.ones((L,), jnp.float32)
                plsc.addupdate_scatter(accum, [i16], v16)
                return 0
            lax.fori_loop(0, blk // L, gbody, 0)

        pltpu.sync_copy(accum, out_hbm.at[wid])

    scratch = [
        pltpu.VMEM((ap,), jnp.float32),
        pltpu.VMEM((blk,), jnp.int32),
        pltpu.VMEM((blk,), jnp.float32),
        pltpu.SemaphoreType.DMA,
    ]
    fn = pl.kernel(
        body,
        out_type=jax.ShapeDtypeStruct((NW, ap), jnp.float32),
        mesh=_mesh(),
        scratch_types=scratch,
        compiler_params=pltpu.CompilerParams(needs_layout_passes=False),
    )
    return fn(idx_p, val_p)[:, :n_out]


@functools.partial(jax.jit, static_argnums=(4, 5, 6))
def _rowscatter_call(x, src_p, dst_p, w_p, n_out, blk, weighted):
    E_pad = src_p.shape[0]
    D = x.shape[1]
    G = 128
    ZC = 112
    r_max = 8500
    npass = _cdiv(n_out, 2 * r_max)
    nr = 2 * npass
    R = _cdiv(n_out, nr)
    zpt = _cdiv(R + 8, NS * ZC) * ZC
    PR = NS * zpt
    trash = R
    nb = E_pad // (NS * blk)
    share = E_pad // NS

    def body(x_hbm, src_hbm, dst_hbm, w_hbm, out_hbm,
             idx_s, dst_s, w_s, srcc, dstc, wc, srcg, dstg, rows_v,
             sem, srcg2, dstg2, rows_v2, sem2, shared):
        cid = lax.axis_index("c")
        sid = lax.axis_index("s")

        def pass_body(p, _p):
            rid = 2 * p + cid
            lo = rid * R
            hi = lo + R

            def zr(r, _):
                for k in range(D // L):
                    rows_v[r, pl.ds(k * L, L)] = jnp.zeros((L,), jnp.float32)
                return 0
            lax.fori_loop(0, ZC, zr, 0)
            for z in range(zpt // ZC):
                pltpu.sync_copy(rows_v.at[pl.ds(0, ZC)],
                                shared.at[pl.ds(sid * zpt + z * ZC, ZC)])
            plsc.subcore_barrier()

            def block_body(bi, _b):
                est = sid * share + bi * blk
                pltpu.sync_copy(src_hbm.at[pl.ds(est, blk)], idx_s)
                pltpu.sync_copy(dst_hbm.at[pl.ds(est, blk)], dst_s)
                if weighted:
                    pltpu.sync_copy(w_hbm.at[pl.ds(est, blk)], w_s)

                def cbody(g, cur):
                    off = g * L
                    d16 = dst_s[pl.ds(off, L)]
                    s16 = idx_s[pl.ds(off, L)]
                    m = (d16 >= lo) & (d16 < hi)
                    plsc.store_compressed(srcc.at[pl.ds(cur, L)], s16, mask=m)
                    plsc.store_compressed(dstc.at[pl.ds(cur, L)], d16 - lo,
                                          mask=m)
                    if weighted:
                        w16 = w_s[pl.ds(off, L)]
                        plsc.store_compressed(wc.at[pl.ds(cur, L)], w16,
                                              mask=m)
                    return cur + jnp.sum(m.astype(jnp.int32))
                cursor = lax.fori_loop(0, blk // L, cbody, jnp.int32(0),
                                       unroll=8)

                for k in range(G // L):
                    srcc[pl.ds(cursor + k * L, L)] = jnp.zeros((L,), jnp.int32)
                    dstc[pl.ds(cursor + k * L, L)] = jnp.full((L,), trash,
                                                              jnp.int32)

                ntrips = (cursor + G - 1) // G

                def stage(j, sg, dg):
                    for k in range(G // L):
                        sg[pl.ds(k * L, L)] = srcc[pl.ds(j * G + k * L, L)]
                        dg[pl.ds(k * L, L)] = dstc[pl.ds(j * G + k * L, L)]

                def process(j, cur_sg, cur_dg, cur_rows, cur_sem,
                            nxt_sg, nxt_dg, nxt_rows, nxt_sem):
                    pltpu.make_async_copy(x_hbm.at[pl.ds(0, G)], cur_rows,
                                          cur_sem).wait()

                    @pl.when(j + 1 < ntrips)
                    def _():
                        stage(j + 1, nxt_sg, nxt_dg)
                        pltpu.async_copy(x_hbm.at[nxt_sg], nxt_rows, nxt_sem)

                    if weighted:
                        def wbody(r, _):
                            wv = wc[pl.ds(j * G + r, L)][0]
                            for k in range(D // L):
                                cur_rows[r, pl.ds(k * L, L)] = (
                                    cur_rows[r, pl.ds(k * L, L)] * wv)
                            return 0
                        lax.fori_loop(0, G, wbody, 0)
                    pltpu.sync_copy(cur_rows, shared.at[cur_dg], add=True)

                @pl.when(ntrips > 0)
                def _():
                    stage(0, srcg, dstg)
                    pltpu.async_copy(x_hbm.at[srcg], rows_v, sem)

                def tbody(j, _):
                    @pl.when(j % 2 == 0)
                    def _():
                        process(j, srcg, dstg, rows_v, sem,
                                srcg2, dstg2, rows_v2, sem2)

                    @pl.when(j % 2 == 1)
                    def _():
                        process(j, srcg2, dstg2, rows_v2, sem2,
                                srcg, dstg, rows_v, sem)
                    return 0
                lax.fori_loop(0, ntrips, tbody, 0)
                return 0
            lax.fori_loop(0, nb, block_body, 0)

            plsc.subcore_barrier()
            for z in range(zpt // ZC):
                roff = sid * zpt + z * ZC
                pltpu.sync_copy(shared.at[pl.ds(roff, ZC)],
                                rows_v.at[pl.ds(0, ZC)])
                pltpu.sync_copy(rows_v.at[pl.ds(0, ZC)],
                                out_hbm.at[pl.ds(rid * PR + roff, ZC)])
            plsc.subcore_barrier()
            return 0
        lax.fori_loop(0, npass, pass_body, 0)

    scratch = [
        pltpu.VMEM((blk,), jnp.int32),
        pltpu.VMEM((blk,), jnp.int32),
        pltpu.VMEM((blk,), jnp.float32),
        pltpu.VMEM((blk + G,), jnp.int32),
        pltpu.VMEM((blk + G,), jnp.int32),
        pltpu.VMEM((blk + G + L,), jnp.float32),
        pltpu.VMEM((G,), jnp.int32),
        pltpu.VMEM((G,), jnp.int32),
        pltpu.VMEM((G, D), jnp.float32),
        pltpu.SemaphoreType.DMA,
        pltpu.VMEM((G,), jnp.int32),
        pltpu.VMEM((G,), jnp.int32),
        pltpu.VMEM((G, D), jnp.float32),
        pltpu.SemaphoreType.DMA,
        pltpu.VMEM_SHARED((PR, D), jnp.float32),
    ]
    fn = pl.kernel(
        body,
        out_type=jax.ShapeDtypeStruct((nr * PR, D), jnp.float32),
        mesh=_mesh(),
        scratch_types=scratch,
        compiler_params=pltpu.CompilerParams(needs_layout_passes=False),
    )
    out_pad = fn(x, src_p, dst_p, w_p)
    return out_pad.reshape(nr, PR, D)[:, :R].reshape(nr * R, D)[:n_out]


def _edge_map_call(tab, idx_p, aux_p, blk, fmap):
    E_pad = idx_p.shape[0]
    n_tab = tab.shape[0]
    ap = _cdiv(n_tab + 1, 128) * 128
    tab = _pad1(tab, ap, 0.0)
    nb = E_pad // (NW * blk)
    share = E_pad // NW

    def body(tab_hbm, idx_hbm, aux_hbm, out_hbm, tab_v, idx_s, aux_s, out_s,
             sem):
        cid = lax.axis_index("c")
        sid = lax.axis_index("s")
        wid = cid * NS + sid
        pltpu.sync_copy(tab_hbm, tab_v)
        for bi in range(nb):
            est = wid * share + bi * blk
            pltpu.sync_copy(idx_hbm.at[pl.ds(est, blk)], idx_s)
            if aux_p is not None:
                pltpu.sync_copy(aux_hbm.at[pl.ds(est, blk)], aux_s)

            def gbody(g, _):
                i16 = idx_s[pl.ds(g * L, L)]
                t16 = plsc.load_gather(tab_v, [i16])
                if aux_p is not None:
                    a16 = aux_s[pl.ds(g * L, L)]
                else:
                    a16 = None
                out_s[pl.ds(g * L, L)] = fmap(t16, a16)
                return 0
            lax.fori_loop(0, blk // L, gbody, 0)
            pltpu.sync_copy(out_s, out_hbm.at[pl.ds(est, blk)])

    scratch = [
        pltpu.VMEM((ap,), jnp.float32),
        pltpu.VMEM((blk,), jnp.int32),
        pltpu.VMEM((blk,), jnp.float32),
        pltpu.VMEM((blk,), jnp.float32),
        pltpu.SemaphoreType.DMA,
    ]
    fn = pl.kernel(
        body,
        out_type=jax.ShapeDtypeStruct((E_pad,), jnp.float32),
        mesh=_mesh(),
        scratch_types=scratch,
        compiler_params=pltpu.CompilerParams(needs_layout_passes=False),
    )
    if aux_p is None:
        aux_p = jnp.zeros((E_pad,), jnp.float32)
    return fn(tab, idx_p, aux_p)


def _norm_from_deg(d):
    return jnp.where(d > 0, lax.rsqrt(jnp.maximum(d, 1e-9)), 0.0)


def _tc1(h_game, dpo, dpi, W0):
    n = h_game.shape[0]
    D = h_game.shape[1]
    nbk = n // _BL

    def fn(h_ref, dpo_ref, dpi_ref, w_ref, y_ref, ns_ref, nd_ref):
        dout = jnp.sum(dpo_ref[...], axis=0)
        din = jnp.sum(dpi_ref[...], axis=0)
        ns = _norm_from_deg(dout)
        nd = _norm_from_deg(din)
        ns_ref[...] = ns.reshape(1, 1, _BL)
        nd_ref[...] = nd.reshape(1, 1, _BL)
        y_ref[...] = jnp.dot(h_ref[...] * ns.reshape(_BL, 1), w_ref[...],
                             preferred_element_type=jnp.float32)

    grid = (nbk,)
    return pl.pallas_call(
        fn,
        grid=grid,
        in_specs=[
            pl.BlockSpec((_BL, D), lambda i: (i, 0)),
            pl.BlockSpec((NW, 1, 1, _BL), lambda i: (0, i, 0, 0)),
            pl.BlockSpec((NW, 1, 1, _BL), lambda i: (0, i, 0, 0)),
            pl.BlockSpec((D, D), lambda i: (0, 0)),
        ],
        out_specs=[
            pl.BlockSpec((_BL, D), lambda i: (i, 0)),
            pl.BlockSpec((1, 1, _BL), lambda i: (i, 0, 0)),
            pl.BlockSpec((1, 1, _BL), lambda i: (i, 0, 0)),
        ],
        out_shape=[
            jax.ShapeDtypeStruct((n, D), jnp.float32),
            jax.ShapeDtypeStruct((nbk, 1, _BL), jnp.float32),
            jax.ShapeDtypeStruct((nbk, 1, _BL), jnp.float32),
        ],
    )(h_game, dpo.reshape(NW, nbk, 1, _BL), dpi.reshape(NW, nbk, 1, _BL), W0)


def _tc3(agg1, ns3, nd3, W1):
    n, D = agg1.shape
    nbk = n // _BL

    def fn(a_ref, ns_ref, nd_ref, w_ref, y_ref):
        ns = ns_ref[...].reshape(_BL, 1)
        nd = nd_ref[...].reshape(_BL, 1)
        y_ref[...] = jnp.dot(a_ref[...] * (nd * ns), w_ref[...],
                             preferred_element_type=jnp.float32)

    return pl.pallas_call(
        fn,
        grid=(nbk,),
        in_specs=[
            pl.BlockSpec((_BL, D), lambda i: (i, 0)),
            pl.BlockSpec((1, 1, _BL), lambda i: (i, 0, 0)),
            pl.BlockSpec((1, 1, _BL), lambda i: (i, 0, 0)),
            pl.BlockSpec((D, D), lambda i: (0, 0)),
        ],
        out_specs=pl.BlockSpec((_BL, D), lambda i: (i, 0)),
        out_shape=jax.ShapeDtypeStruct((n, D), jnp.float32),
    )(agg1, ns3, nd3, W1)


def _tc5(agg2, nd3):
    n, D = agg2.shape
    nbk = n // _BL

    def fn(a_ref, nd_ref, y_ref):
        y_ref[...] = a_ref[...] * nd_ref[...].reshape(_BL, 1)

    return pl.pallas_call(
        fn,
        grid=(nbk,),
        in_specs=[
            pl.BlockSpec((_BL, D), lambda i: (i, 0)),
            pl.BlockSpec((1, 1, _BL), lambda i: (i, 0, 0)),
        ],
        out_specs=pl.BlockSpec((_BL, D), lambda i: (i, 0)),
        out_shape=jax.ShapeDtypeStruct((n, D), jnp.float32),
    )(agg2, nd3)


def _tc7(ue, sgu, cgu, cso, Wsi, Wni, bi, Wg, al, ar):
    n, D = ue.shape
    nbk = n // _BL

    def fn(ue_ref, sgu_ref, cgu_ref, cso_ref, wsi_ref, wni_ref, bi_ref,
           wg_ref, al_ref, ar_ref, hua_ref, el_ref, er_ref, ivs_ref):
        cg = jnp.sum(cgu_ref[...], axis=0)
        cs = jnp.sum(cso_ref[...], axis=0)
        ivg = 1.0 / jnp.maximum(cg, 1.0)
        ivs = 1.0 / jnp.maximum(cs, 1.0)
        ivs_ref[...] = ivs.reshape(1, 1, _BL)
        hua = (jnp.dot(ue_ref[...], wsi_ref[...],
                       preferred_element_type=jnp.float32)
               + jnp.dot(sgu_ref[...], wni_ref[...],
                         preferred_element_type=jnp.float32)
               * ivg.reshape(_BL, 1)
               + bi_ref[...])
        hua_ref[...] = hua
        feat = jnp.dot(hua, wg_ref[...], preferred_element_type=jnp.float32)
        el = jnp.sum(feat * al_ref[...], axis=1)
        er = jnp.sum(feat * ar_ref[...], axis=1)
        el_ref[...] = el.reshape(1, 1, _BL)
        er_ref[...] = er.reshape(1, 1, _BL)

    return pl.pallas_call(
        fn,
        grid=(nbk,),
        in_specs=[
            pl.BlockSpec((_BL, D), lambda i: (i, 0)),
            pl.BlockSpec((_BL, D), lambda i: (i, 0)),
            pl.BlockSpec((NW, 1, 1, _BL), lambda i: (0, i, 0, 0)),
            pl.BlockSpec((NW, 1, 1, _BL), lambda i: (0, i, 0, 0)),
            pl.BlockSpec((D, D), lambda i: (0, 0)),
            pl.BlockSpec((D, D), lambda i: (0, 0)),
            pl.BlockSpec((1, D), lambda i: (0, 0)),
            pl.BlockSpec((D, D), lambda i: (0, 0)),
            pl.BlockSpec((1, D), lambda i: (0, 0)),
            pl.BlockSpec((1, D), lambda i: (0, 0)),
        ],
        out_specs=[
            pl.BlockSpec((_BL, D), lambda i: (i, 0)),
            pl.BlockSpec((1, 1, _BL), lambda i: (i, 0, 0)),
            pl.BlockSpec((1, 1, _BL), lambda i: (i, 0, 0)),
            pl.BlockSpec((1, 1, _BL), lambda i: (i, 0, 0)),
        ],
        out_shape=[
            jax.ShapeDtypeStruct((n, D), jnp.float32),
            jax.ShapeDtypeStruct((nbk, 1, _BL), jnp.float32),
            jax.ShapeDtypeStruct((nbk, 1, _BL), jnp.float32),
            jax.ShapeDtypeStruct((nbk, 1, _BL), jnp.float32),
        ],
    )(ue, sgu, cgu.reshape(NW, nbk, 1, _BL), cso.reshape(NW, nbk, 1, _BL),
      Wsi, Wni, bi.reshape(1, D), Wg, al.reshape(1, D), ar.reshape(1, D))


def _tc11(esp):
    nw, n = esp.shape
    nbk = n // _BL

    def fn(p_ref, o_ref):
        o_ref[...] = jnp.sum(p_ref[...], axis=0).reshape(1, 1, _BL)

    return pl.pallas_call(
        fn,
        grid=(nbk,),
        in_specs=[pl.BlockSpec((NW, 1, 1, _BL), lambda i: (0, i, 0, 0))],
        out_specs=pl.BlockSpec((1, 1, _BL), lambda i: (i, 0, 0)),
        out_shape=jax.ShapeDtypeStruct((nbk, 1, _BL), jnp.float32),
    )(esp.reshape(NW, nbk, 1, _BL))


def _tc14(ue, hua, ssoc, ivs3, Wss, Wns, bs, wu, wa, ws):
    n, D = ue.shape
    nbk = n // _BL

    def fn(ue_ref, hua_ref, ss_ref, ivs_ref, wss_ref, wns_ref, bs_ref, o_ref):
        ivs = ivs_ref[...].reshape(_BL, 1)
        hus = (jnp.dot(ue_ref[...], wss_ref[...],
                       preferred_element_type=jnp.float32)
               + jnp.dot(ss_ref[...], wns_ref[...],
                         preferred_element_type=jnp.float32) * ivs
               + bs_ref[...])
        o_ref[...] = wu * ue_ref[...] + wa * hua_ref[...] + ws * hus

    return pl.pallas_call(
        fn,
        grid=(nbk,),
        in_specs=[
            pl.BlockSpec((_BL, D), lambda i: (i, 0)),
            pl.BlockSpec((_BL, D), lambda i: (i, 0)),
            pl.BlockSpec((_BL, D), lambda i: (i, 0)),
            pl.BlockSpec((1, 1, _BL), lambda i: (i, 0, 0)),
            pl.BlockSpec((D, D), lambda i: (0, 0)),
            pl.BlockSpec((D, D), lambda i: (0, 0)),
            pl.BlockSpec((1, D), lambda i: (0, 0)),
        ],
        out_specs=pl.BlockSpec((_BL, D), lambda i: (i, 0)),
        out_shape=jax.ShapeDtypeStruct((n, D), jnp.float32),
    )(ue, hua, ssoc, ivs3, Wss, Wns, bs.reshape(1, D))


def kernel(h_game, gu_weight, user_embedding, item_embedding, W_gc0, W_gc1,
           W_self_i, W_neigh_i, b_i, W_gat, attn_l, attn_r, W_self_s,
           W_neigh_s, b_s, item_edge_index, gu_src, gu_dst,
           social_edge_index):
    ni = h_game.shape[0]
    nu = user_embedding.shape[0]
    ei = item_edge_index.shape[1]
    eg = gu_src.shape[0]
    es = social_edge_index.shape[1]

    iblk, ei_pad = _pad_edges(ei)
    gblk, eg_pad = _pad_edges(eg)
    sblk, es_pad = _pad_edges(es)

    isrc = _pad1(item_edge_index[0], ei_pad, 0)
    idst = _pad1(item_edge_index[1], ei_pad, ni)
    gus = _pad1(gu_src, eg_pad, 0)
    gud = _pad1(gu_dst, eg_pad, nu)
    guw = _pad1(gu_weight, eg_pad, 0.0)
    ssrc = _pad1(social_edge_index[0], es_pad, 0)
    sdst = _pad1(social_edge_index[1], es_pad, nu)

    zi = jnp.zeros((ei_pad,), jnp.float32)
    zg = jnp.zeros((eg_pad,), jnp.float32)
    zs = jnp.zeros((es_pad,), jnp.float32)

    dpo = _hist_call(isrc, zi, ni, iblk, False)
    dpi = _hist_call(idst, zi, ni, iblk, False)
    cgu = _hist_call(gud, zg, nu, gblk, False)
    cso = _hist_call(sdst, zs, nu, sblk, False)

    y0, ns3, nd3 = _tc1(h_game, dpo, dpi, W_gc0)
    agg1 = _rowscatter_call(y0, isrc, idst, zi, ni, iblk, False)
    y1 = _tc3(agg1, ns3, nd3, W_gc1)
    agg2 = _rowscatter_call(y1, isrc, idst, zi, ni, iblk, False)
    h2 = _tc5(agg2, nd3)

    sgu = _rowscatter_call(h2, gus, gud, guw, nu, gblk, True)
    hua, el3, er3, ivs3 = _tc7(user_embedding, sgu, cgu, cso, W_self_i,
                               W_neigh_i, b_i, W_gat, attn_l, attn_r)

    el = el3.reshape(nu)
    er = er3.reshape(nu)
    tmp = _edge_map_call(el, ssrc, None, sblk, lambda t, a: t)

    def _exp_leaky(t, a):
        e = a + t
        e = jnp.where(e >= 0, e, 0.2 * e)
        return jnp.exp(e)
    ex = _edge_map_call(er, sdst, tmp, sblk, _exp_leaky)
    esp = _hist_call(sdst, ex, nu, sblk, True)
    esum3 = _tc11(esp)
    alpha = _edge_map_call(esum3.reshape(nu), sdst, ex, sblk,
                           lambda t, a: a / (t + 1e-9))

    ssoc = _rowscatter_call(user_embedding, ssrc, sdst, alpha, nu, sblk, True)

    wu = 1.0 - 0.1 - 0.2
    user_out = _tc14(user_embedding, hua, ssoc, ivs3, W_self_s, W_neigh_s,
                     b_s, wu, 0.2, 0.1)
    return user_out, item_embedding


# speedup vs baseline: 1.0987x; 1.0987x over previous
import functools
import jax
import jax.numpy as jnp
from jax import lax
from jax.experimental import pallas as pl
from jax.experimental.pallas import tpu as pltpu
from jax.experimental.pallas import tpu_sc as plsc

NC = 2
NS = 16
NW = NC * NS
L = 16

_BL = 400


def _cdiv(a, b):
    return -(-a // b)


def _mesh():
    return plsc.VectorSubcoreMesh(core_axis_name="c", subcore_axis_name="s",
                                  num_cores=NC, num_subcores=NS)


def _pad_edges(E):
    blk = min(4096, _cdiv(E, NW * L) * L)
    nb = _cdiv(E, NW * blk)
    return blk, NW * blk * nb


def _pad1(x, n, val):
    if x.shape[0] == n:
        return x
    return jnp.concatenate(
        [x, jnp.full((n - x.shape[0],), val, x.dtype)])


@functools.partial(jax.jit, static_argnums=(2, 3, 4))
def _hist_call(idx_p, val_p, n_out, blk, weighted):
    E_pad = idx_p.shape[0]
    ap = _cdiv(n_out + 1, 128) * 128
    nb = E_pad // (NW * blk)
    share = E_pad // NW

    def body(idx_hbm, val_hbm, out_hbm, accum, idx_s, val_s, sem):
        cid = lax.axis_index("c")
        sid = lax.axis_index("s")
        wid = cid * NS + sid

        def zbody(z, _):
            accum[pl.ds(z * L, L)] = jnp.zeros((L,), jnp.float32)
            return 0
        lax.fori_loop(0, ap // L, zbody, 0)

        for bi in range(nb):
            est = wid * share + bi * blk
            pltpu.sync_copy(idx_hbm.at[pl.ds(est, blk)], idx_s)
            if weighted:
                pltpu.sync_copy(val_hbm.at[pl.ds(est, blk)], val_s)

            def gbody(g, _):
                i16 = idx_s[pl.ds(g * L, L)]
                if weighted:
                    v16 = val_s[pl.ds(g * L, L)]
                else:
                    v16 = jnp.ones((L,), jnp.float32)
                plsc.addupdate_scatter(accum, [i16], v16)
                return 0
            lax.fori_loop(0, blk // L, gbody, 0)

        pltpu.sync_copy(accum, out_hbm.at[wid])

    scratch = [
        pltpu.VMEM((ap,), jnp.float32),
        pltpu.VMEM((blk,), jnp.int32),
        pltpu.VMEM((blk,), jnp.float32),
        pltpu.SemaphoreType.DMA,
    ]
    fn = pl.kernel(
        body,
        out_type=jax.ShapeDtypeStruct((NW, ap), jnp.float32),
        mesh=_mesh(),
        scratch_types=scratch,
        compiler_params=pltpu.CompilerParams(needs_layout_passes=False),
    )
    return fn(idx_p, val_p)[:, :n_out]


@functools.partial(jax.jit, static_argnums=(4, 5, 6))
def _rowscatter_call(x, src_p, dst_p, w_p, n_out, blk, weighted):
    E_pad = src_p.shape[0]
    D = x.shape[1]
    G = 128
    ZC = 112
    r_max = 10000
    npass = _cdiv(n_out, 2 * r_max)
    nr = 2 * npass
    R = _cdiv(n_out, nr)
    zpt = _cdiv(R + 8, NS * ZC) * ZC
    PR = NS * zpt
    trash = R
    nb = E_pad // (NS * blk)
    share = E_pad // NS

    def body(x_hbm, src_hbm, dst_hbm, w_hbm, out_hbm,
             idx_s, dst_s, w_s, srcc, dstc, wc, srcg, dstg, rows_v,
             sem, shared):
        cid = lax.axis_index("c")
        sid = lax.axis_index("s")

        for p in range(npass):
            rid = 2 * p + cid
            lo = rid * R
            hi = lo + R

            def zr(r, _):
                for k in range(D // L):
                    rows_v[r, pl.ds(k * L, L)] = jnp.zeros((L,), jnp.float32)
                return 0
            lax.fori_loop(0, ZC, zr, 0)
            for z in range(zpt // ZC):
                pltpu.sync_copy(rows_v.at[pl.ds(0, ZC)],
                                shared.at[pl.ds(sid * zpt + z * ZC, ZC)])
            plsc.subcore_barrier()

            def block_body(bi, _b):
                est = sid * share + bi * blk
                pltpu.sync_copy(src_hbm.at[pl.ds(est, blk)], idx_s)
                pltpu.sync_copy(dst_hbm.at[pl.ds(est, blk)], dst_s)
                if weighted:
                    pltpu.sync_copy(w_hbm.at[pl.ds(est, blk)], w_s)

                def cbody(g, cur):
                    off = g * L
                    d16 = dst_s[pl.ds(off, L)]
                    s16 = idx_s[pl.ds(off, L)]
                    m = (d16 >= lo) & (d16 < hi)
                    plsc.store_compressed(srcc.at[pl.ds(cur, L)], s16, mask=m)
                    plsc.store_compressed(dstc.at[pl.ds(cur, L)], d16 - lo,
                                          mask=m)
                    if weighted:
                        w16 = w_s[pl.ds(off, L)]
                        plsc.store_compressed(wc.at[pl.ds(cur, L)], w16,
                                              mask=m)
                    return cur + jnp.sum(m.astype(jnp.int32))
                cursor = lax.fori_loop(0, blk // L, cbody, jnp.int32(0),
                                       unroll=4)

                for k in range(G // L):
                    srcc[pl.ds(cursor + k * L, L)] = jnp.zeros((L,), jnp.int32)
                    dstc[pl.ds(cursor + k * L, L)] = jnp.full((L,), trash,
                                                              jnp.int32)

                def tbody(j, _):
                    for k in range(G // L):
                        srcg[pl.ds(k * L, L)] = srcc[pl.ds(j * G + k * L, L)]
                        dstg[pl.ds(k * L, L)] = dstc[pl.ds(j * G + k * L, L)]
                    pltpu.async_copy(x_hbm.at[srcg], rows_v, sem).wait()
                    if weighted:
                        def wbody(r, _):
                            wv = wc[pl.ds(j * G + r, L)][0]
                            for k in range(D // L):
                                rows_v[r, pl.ds(k * L, L)] = (
                                    rows_v[r, pl.ds(k * L, L)] * wv)
                            return 0
                        lax.fori_loop(0, G, wbody, 0)
                    pltpu.sync_copy(rows_v, shared.at[dstg], add=True)
                    return 0
                ntrips = (cursor + G - 1) // G
                lax.fori_loop(0, ntrips, tbody, 0)
                return 0
            lax.fori_loop(0, nb, block_body, 0)

            plsc.subcore_barrier()
            for z in range(zpt // ZC):
                roff = sid * zpt + z * ZC
                pltpu.sync_copy(shared.at[pl.ds(roff, ZC)],
                                rows_v.at[pl.ds(0, ZC)])
                pltpu.sync_copy(rows_v.at[pl.ds(0, ZC)],
                                out_hbm.at[pl.ds(rid * PR + roff, ZC)])
            plsc.subcore_barrier()

    scratch = [
        pltpu.VMEM((blk,), jnp.int32),
        pltpu.VMEM((blk,), jnp.int32),
        pltpu.VMEM((blk,), jnp.float32),
        pltpu.VMEM((blk + G,), jnp.int32),
        pltpu.VMEM((blk + G,), jnp.int32),
        pltpu.VMEM((blk + G + L,), jnp.float32),
        pltpu.VMEM((G,), jnp.int32),
        pltpu.VMEM((G,), jnp.int32),
        pltpu.VMEM((G, D), jnp.float32),
        pltpu.SemaphoreType.DMA,
        pltpu.VMEM_SHARED((PR, D), jnp.float32),
    ]
    fn = pl.kernel(
        body,
        out_type=jax.ShapeDtypeStruct((nr * PR, D), jnp.float32),
        mesh=_mesh(),
        scratch_types=scratch,
        compiler_params=pltpu.CompilerParams(needs_layout_passes=False),
    )
    out_pad = fn(x, src_p, dst_p, w_p)
    return out_pad.reshape(nr, PR, D)[:, :R].reshape(nr * R, D)[:n_out]


def _edge_map_call(tab, idx_p, aux_p, blk, fmap):
    E_pad = idx_p.shape[0]
    n_tab = tab.shape[0]
    ap = _cdiv(n_tab + 1, 128) * 128
    tab = _pad1(tab, ap, 0.0)
    nb = E_pad // (NW * blk)
    share = E_pad // NW

    def body(tab_hbm, idx_hbm, aux_hbm, out_hbm, tab_v, idx_s, aux_s, out_s,
             sem):
        cid = lax.axis_index("c")
        sid = lax.axis_index("s")
        wid = cid * NS + sid
        pltpu.sync_copy(tab_hbm, tab_v)
        for bi in range(nb):
            est = wid * share + bi * blk
            pltpu.sync_copy(idx_hbm.at[pl.ds(est, blk)], idx_s)
            if aux_p is not None:
                pltpu.sync_copy(aux_hbm.at[pl.ds(est, blk)], aux_s)

            def gbody(g, _):
                i16 = idx_s[pl.ds(g * L, L)]
                t16 = plsc.load_gather(tab_v, [i16])
                if aux_p is not None:
                    a16 = aux_s[pl.ds(g * L, L)]
                else:
                    a16 = None
                out_s[pl.ds(g * L, L)] = fmap(t16, a16)
                return 0
            lax.fori_loop(0, blk // L, gbody, 0)
            pltpu.sync_copy(out_s, out_hbm.at[pl.ds(est, blk)])

    scratch = [
        pltpu.VMEM((ap,), jnp.float32),
        pltpu.VMEM((blk,), jnp.int32),
        pltpu.VMEM((blk,), jnp.float32),
        pltpu.VMEM((blk,), jnp.float32),
        pltpu.SemaphoreType.DMA,
    ]
    fn = pl.kernel(
        body,
        out_type=jax.ShapeDtypeStruct((E_pad,), jnp.float32),
        mesh=_mesh(),
        scratch_types=scratch,
        compiler_params=pltpu.CompilerParams(needs_layout_passes=False),
    )
    if aux_p is None:
        aux_p = jnp.zeros((E_pad,), jnp.float32)
    return fn(tab, idx_p, aux_p)


def _norm_from_deg(d):
    return jnp.where(d > 0, lax.rsqrt(jnp.maximum(d, 1e-9)), 0.0)


def _tc1(h_game, dpo, dpi, W0):
    n = h_game.shape[0]
    D = h_game.shape[1]
    nbk = n // _BL

    def fn(h_ref, dpo_ref, dpi_ref, w_ref, y_ref, ns_ref, nd_ref):
        dout = jnp.sum(dpo_ref[...], axis=0)
        din = jnp.sum(dpi_ref[...], axis=0)
        ns = _norm_from_deg(dout)
        nd = _norm_from_deg(din)
        ns_ref[...] = ns.reshape(1, 1, _BL)
        nd_ref[...] = nd.reshape(1, 1, _BL)
        y_ref[...] = jnp.dot(h_ref[...] * ns.reshape(_BL, 1), w_ref[...],
                             preferred_element_type=jnp.float32)

    grid = (nbk,)
    return pl.pallas_call(
        fn,
        grid=grid,
        in_specs=[
            pl.BlockSpec((_BL, D), lambda i: (i, 0)),
            pl.BlockSpec((NW, 1, 1, _BL), lambda i: (0, i, 0, 0)),
            pl.BlockSpec((NW, 1, 1, _BL), lambda i: (0, i, 0, 0)),
            pl.BlockSpec((D, D), lambda i: (0, 0)),
        ],
        out_specs=[
            pl.BlockSpec((_BL, D), lambda i: (i, 0)),
            pl.BlockSpec((1, 1, _BL), lambda i: (i, 0, 0)),
            pl.BlockSpec((1, 1, _BL), lambda i: (i, 0, 0)),
        ],
        out_shape=[
            jax.ShapeDtypeStruct((n, D), jnp.float32),
            jax.ShapeDtypeStruct((nbk, 1, _BL), jnp.float32),
            jax.ShapeDtypeStruct((nbk, 1, _BL), jnp.float32),
        ],
    )(h_game, dpo.reshape(NW, nbk, 1, _BL), dpi.reshape(NW, nbk, 1, _BL), W0)


def _tc3(agg1, ns3, nd3, W1):
    n, D = agg1.shape
    nbk = n // _BL

    def fn(a_ref, ns_ref, nd_ref, w_ref, y_ref):
        ns = ns_ref[...].reshape(_BL, 1)
        nd = nd_ref[...].reshape(_BL, 1)
        y_ref[...] = jnp.dot(a_ref[...] * (nd * ns), w_ref[...],
                             preferred_element_type=jnp.float32)

    return pl.pallas_call(
        fn,
        grid=(nbk,),
        in_specs=[
            pl.BlockSpec((_BL, D), lambda i: (i, 0)),
            pl.BlockSpec((1, 1, _BL), lambda i: (i, 0, 0)),
            pl.BlockSpec((1, 1, _BL), lambda i: (i, 0, 0)),
            pl.BlockSpec((D, D), lambda i: (0, 0)),
        ],
        out_specs=pl.BlockSpec((_BL, D), lambda i: (i, 0)),
        out_shape=jax.ShapeDtypeStruct((n, D), jnp.float32),
    )(agg1, ns3, nd3, W1)


def _tc5(agg2, nd3):
    n, D = agg2.shape
    nbk = n // _BL

    def fn(a_ref, nd_ref, y_ref):
        y_ref[...] = a_ref[...] * nd_ref[...].reshape(_BL, 1)

    return pl.pallas_call(
        fn,
        grid=(nbk,),
        in_specs=[
            pl.BlockSpec((_BL, D), lambda i: (i, 0)),
            pl.BlockSpec((1, 1, _BL), lambda i: (i, 0, 0)),
        ],
        out_specs=pl.BlockSpec((_BL, D), lambda i: (i, 0)),
        out_shape=jax.ShapeDtypeStruct((n, D), jnp.float32),
    )(agg2, nd3)


def _tc7(ue, sgu, cgu, cso, Wsi, Wni, bi, Wg, al, ar):
    n, D = ue.shape
    nbk = n // _BL

    def fn(ue_ref, sgu_ref, cgu_ref, cso_ref, wsi_ref, wni_ref, bi_ref,
           wg_ref, al_ref, ar_ref, hua_ref, el_ref, er_ref, ivs_ref):
        cg = jnp.sum(cgu_ref[...], axis=0)
        cs = jnp.sum(cso_ref[...], axis=0)
        ivg = 1.0 / jnp.maximum(cg, 1.0)
        ivs = 1.0 / jnp.maximum(cs, 1.0)
        ivs_ref[...] = ivs.reshape(1, 1, _BL)
        hua = (jnp.dot(ue_ref[...], wsi_ref[...],
                       preferred_element_type=jnp.float32)
               + jnp.dot(sgu_ref[...], wni_ref[...],
                         preferred_element_type=jnp.float32)
               * ivg.reshape(_BL, 1)
               + bi_ref[...])
        hua_ref[...] = hua
        feat = jnp.dot(hua, wg_ref[...], preferred_element_type=jnp.float32)
        el = jnp.sum(feat * al_ref[...], axis=1)
        er = jnp.sum(feat * ar_ref[...], axis=1)
        el_ref[...] = el.reshape(1, 1, _BL)
        er_ref[...] = er.reshape(1, 1, _BL)

    return pl.pallas_call(
        fn,
        grid=(nbk,),
        in_specs=[
            pl.BlockSpec((_BL, D), lambda i: (i, 0)),
            pl.BlockSpec((_BL, D), lambda i: (i, 0)),
            pl.BlockSpec((NW, 1, 1, _BL), lambda i: (0, i, 0, 0)),
            pl.BlockSpec((NW, 1, 1, _BL), lambda i: (0, i, 0, 0)),
            pl.BlockSpec((D, D), lambda i: (0, 0)),
            pl.BlockSpec((D, D), lambda i: (0, 0)),
            pl.BlockSpec((1, D), lambda i: (0, 0)),
            pl.BlockSpec((D, D), lambda i: (0, 0)),
            pl.BlockSpec((1, D), lambda i: (0, 0)),
            pl.BlockSpec((1, D), lambda i: (0, 0)),
        ],
        out_specs=[
            pl.BlockSpec((_BL, D), lambda i: (i, 0)),
            pl.BlockSpec((1, 1, _BL), lambda i: (i, 0, 0)),
            pl.BlockSpec((1, 1, _BL), lambda i: (i, 0, 0)),
            pl.BlockSpec((1, 1, _BL), lambda i: (i, 0, 0)),
        ],
        out_shape=[
            jax.ShapeDtypeStruct((n, D), jnp.float32),
            jax.ShapeDtypeStruct((nbk, 1, _BL), jnp.float32),
            jax.ShapeDtypeStruct((nbk, 1, _BL), jnp.float32),
            jax.ShapeDtypeStruct((nbk, 1, _BL), jnp.float32),
        ],
    )(ue, sgu, cgu.reshape(NW, nbk, 1, _BL), cso.reshape(NW, nbk, 1, _BL),
      Wsi, Wni, bi.reshape(1, D), Wg, al.reshape(1, D), ar.reshape(1, D))


def _tc11(esp):
    nw, n = esp.shape
    nbk = n // _BL

    def fn(p_ref, o_ref):
        o_ref[...] = jnp.sum(p_ref[...], axis=0).reshape(1, 1, _BL)

    return pl.pallas_call(
        fn,
        grid=(nbk,),
        in_specs=[pl.BlockSpec((NW, 1, 1, _BL), lambda i: (0, i, 0, 0))],
        out_specs=pl.BlockSpec((1, 1, _BL), lambda i: (i, 0, 0)),
        out_shape=jax.ShapeDtypeStruct((nbk, 1, _BL), jnp.float32),
    )(esp.reshape(NW, nbk, 1, _BL))


def _tc14(ue, hua, ssoc, ivs3, Wss, Wns, bs, wu, wa, ws):
    n, D = ue.shape
    nbk = n // _BL

    def fn(ue_ref, hua_ref, ss_ref, ivs_ref, wss_ref, wns_ref, bs_ref, o_ref):
        ivs = ivs_ref[...].reshape(_BL, 1)
        hus = (jnp.dot(ue_ref[...], wss_ref[...],
                       preferred_element_type=jnp.float32)
               + jnp.dot(ss_ref[...], wns_ref[...],
                         preferred_element_type=jnp.float32) * ivs
               + bs_ref[...])
        o_ref[...] = wu * ue_ref[...] + wa * hua_ref[...] + ws * hus

    return pl.pallas_call(
        fn,
        grid=(nbk,),
        in_specs=[
            pl.BlockSpec((_BL, D), lambda i: (i, 0)),
            pl.BlockSpec((_BL, D), lambda i: (i, 0)),
            pl.BlockSpec((_BL, D), lambda i: (i, 0)),
            pl.BlockSpec((1, 1, _BL), lambda i: (i, 0, 0)),
            pl.BlockSpec((D, D), lambda i: (0, 0)),
            pl.BlockSpec((D, D), lambda i: (0, 0)),
            pl.BlockSpec((1, D), lambda i: (0, 0)),
        ],
        out_specs=pl.BlockSpec((_BL, D), lambda i: (i, 0)),
        out_shape=jax.ShapeDtypeStruct((n, D), jnp.float32),
    )(ue, hua, ssoc, ivs3, Wss, Wns, bs.reshape(1, D))


def kernel(h_game, gu_weight, user_embedding, item_embedding, W_gc0, W_gc1,
           W_self_i, W_neigh_i, b_i, W_gat, attn_l, attn_r, W_self_s,
           W_neigh_s, b_s, item_edge_index, gu_src, gu_dst,
           social_edge_index):
    ni = h_game.shape[0]
    nu = user_embedding.shape[0]
    ei = item_edge_index.shape[1]
    eg = gu_src.shape[0]
    es = social_edge_index.shape[1]

    iblk, ei_pad = _pad_edges(ei)
    gblk, eg_pad = _pad_edges(eg)
    sblk, es_pad = _pad_edges(es)

    isrc = _pad1(item_edge_index[0], ei_pad, 0)
    idst = _pad1(item_edge_index[1], ei_pad, ni)
    gus = _pad1(gu_src, eg_pad, 0)
    gud = _pad1(gu_dst, eg_pad, nu)
    guw = _pad1(gu_weight, eg_pad, 0.0)
    ssrc = _pad1(social_edge_index[0], es_pad, 0)
    sdst = _pad1(social_edge_index[1], es_pad, nu)

    zi = jnp.zeros((ei_pad,), jnp.float32)
    zg = jnp.zeros((eg_pad,), jnp.float32)
    zs = jnp.zeros((es_pad,), jnp.float32)

    dpo = _hist_call(isrc, zi, ni, iblk, False)
    dpi = _hist_call(idst, zi, ni, iblk, False)
    cgu = _hist_call(gud, zg, nu, gblk, False)
    cso = _hist_call(sdst, zs, nu, sblk, False)

    y0, ns3, nd3 = _tc1(h_game, dpo, dpi, W_gc0)
    agg1 = _rowscatter_call(y0, isrc, idst, zi, ni, iblk, False)
    y1 = _tc3(agg1, ns3, nd3, W_gc1)
    agg2 = _rowscatter_call(y1, isrc, idst, zi, ni, iblk, False)
    h2 = _tc5(agg2, nd3)

    sgu = _rowscatter_call(h2, gus, gud, guw, nu, gblk, True)
    hua, el3, er3, ivs3 = _tc7(user_embedding, sgu, cgu, cso, W_self_i,
                               W_neigh_i, b_i, W_gat, attn_l, attn_r)

    el = el3.reshape(nu)
    er = er3.reshape(nu)
    tmp = _edge_map_call(el, ssrc, None, sblk, lambda t, a: t)

    def _exp_leaky(t, a):
        e = a + t
        e = jnp.where(e >= 0, e, 0.2 * e)
        return jnp.exp(e)
    ex = _edge_map_call(er, sdst, tmp, sblk, _exp_leaky)
    esp = _hist_call(sdst, ex, nu, sblk, True)
    esum3 = _tc11(esp)
    alpha = _edge_map_call(esum3.reshape(nu), sdst, ex, sblk,
                           lambda t, a: a / (t + 1e-9))

    ssoc = _rowscatter_call(user_embedding, ssrc, sdst, alpha, nu, sblk, True)

    wu = 1.0 - 0.1 - 0.2
    user_out = _tc14(user_embedding, hua, ssoc, ivs3, W_self_s, W_neigh_s,
                     b_s, wu, 0.2, 0.1)
    return user_out, item_embedding


# speedup vs baseline: 2.8875x; 2.6281x over previous
import functools
import jax
import jax.numpy as jnp
from jax import lax
from jax.experimental import pallas as pl
from jax.experimental.pallas import tpu as pltpu
from jax.experimental.pallas import tpu_sc as plsc

NC = 2
NS = 16
NW = NC * NS
L = 16

_BL = 400


def _cdiv(a, b):
    return -(-a // b)


def _mesh():
    return plsc.VectorSubcoreMesh(core_axis_name="c", subcore_axis_name="s",
                                  num_cores=NC, num_subcores=NS)


def _pad_edges(E):
    blk = min(4096, _cdiv(E, NW * L) * L)
    nb = _cdiv(E, NW * blk)
    return blk, NW * blk * nb


def _pad1(x, n, val):
    if x.shape[0] == n:
        return x
    return jnp.concatenate(
        [x, jnp.full((n - x.shape[0],), val, x.dtype)])


@functools.partial(jax.jit, static_argnums=(2, 3, 4))
def _hist_call(idx_p, val_p, n_out, blk, weighted):
    E_pad = idx_p.shape[0]
    ap = _cdiv(n_out + 1, 128) * 128
    nb = E_pad // (NW * blk)
    share = E_pad // NW

    def body(idx_hbm, val_hbm, out_hbm, accum, idx_s, val_s, sem):
        cid = lax.axis_index("c")
        sid = lax.axis_index("s")
        wid = cid * NS + sid

        def zbody(z, _):
            accum[pl.ds(z * L, L)] = jnp.zeros((L,), jnp.float32)
            return 0
        lax.fori_loop(0, ap // L, zbody, 0)

        for bi in range(nb):
            est = wid * share + bi * blk
            pltpu.sync_copy(idx_hbm.at[pl.ds(est, blk)], idx_s)
            if weighted:
                pltpu.sync_copy(val_hbm.at[pl.ds(est, blk)], val_s)

            def gbody(g, _):
                i16 = idx_s[pl.ds(g * L, L)]
                if weighted:
                    v16 = val_s[pl.ds(g * L, L)]
                else:
                    v16 = jnp.ones((L,), jnp.float32)
                plsc.addupdate_scatter(accum, [i16], v16)
                return 0
            lax.fori_loop(0, blk // L, gbody, 0)

        pltpu.sync_copy(accum, out_hbm.at[wid])

    scratch = [
        pltpu.VMEM((ap,), jnp.float32),
        pltpu.VMEM((blk,), jnp.int32),
        pltpu.VMEM((blk,), jnp.float32),
        pltpu.SemaphoreType.DMA,
    ]
    fn = pl.kernel(
        body,
        out_type=jax.ShapeDtypeStruct((NW, ap), jnp.float32),
        mesh=_mesh(),
        scratch_types=scratch,
        compiler_params=pltpu.CompilerParams(needs_layout_passes=False),
    )
    return fn(idx_p, val_p)[:, :n_out]


@functools.partial(jax.jit, static_argnums=(4, 5, 6))
def _rowscatter_call(x, src_p, dst_p, w_p, n_out, blk, weighted):
    E_pad = src_p.shape[0]
    D = x.shape[1]
    G = 128
    ZC = 112
    r_max = 10000
    npass = _cdiv(n_out, 2 * r_max)
    nr = 2 * npass
    R = _cdiv(n_out, nr)
    zpt = _cdiv(R + 8, NS * ZC) * ZC
    PR = NS * zpt
    trash = R
    nb = E_pad // (NS * blk)
    share = E_pad // NS

    def body(x_hbm, src_hbm, dst_hbm, w_hbm, out_hbm,
             idx_s, dst_s, w_s, srcc, dstc, wc, srcg, dstg, rows_v,
             sem, shared):
        cid = lax.axis_index("c")
        sid = lax.axis_index("s")

        for p in range(npass):
            rid = 2 * p + cid
            lo = rid * R
            hi = lo + R

            def zr(r, _):
                for k in range(D // L):
                    rows_v[r, pl.ds(k * L, L)] = jnp.zeros((L,), jnp.float32)
                return 0
            lax.fori_loop(0, ZC, zr, 0)
            for z in range(zpt // ZC):
                pltpu.sync_copy(rows_v.at[pl.ds(0, ZC)],
                                shared.at[pl.ds(sid * zpt + z * ZC, ZC)])
            plsc.subcore_barrier()

            def tbody(j, _):
                for k in range(G // L):
                    srcg[pl.ds(k * L, L)] = srcc[pl.ds(j * G + k * L, L)]
                    dstg[pl.ds(k * L, L)] = dstc[pl.ds(j * G + k * L, L)]
                pltpu.async_copy(x_hbm.at[srcg], rows_v, sem).wait()
                if weighted:
                    def wbody(r, _):
                        wv = wc[pl.ds(j * G + r, L)][0]
                        for k in range(D // L):
                            rows_v[r, pl.ds(k * L, L)] = (
                                rows_v[r, pl.ds(k * L, L)] * wv)
                        return 0
                    lax.fori_loop(0, G, wbody, 0, unroll=2)
                pltpu.sync_copy(rows_v, shared.at[dstg], add=True)
                return 0

            def block_body(bi, cur0):
                est = sid * share + bi * blk
                pltpu.sync_copy(src_hbm.at[pl.ds(est, blk)], idx_s)
                pltpu.sync_copy(dst_hbm.at[pl.ds(est, blk)], dst_s)
                if weighted:
                    pltpu.sync_copy(w_hbm.at[pl.ds(est, blk)], w_s)

                def cbody(g, cur):
                    off = g * L
                    d16 = dst_s[pl.ds(off, L)]
                    s16 = idx_s[pl.ds(off, L)]
                    m = (d16 >= lo) & (d16 < hi)
                    plsc.store_compressed(srcc.at[pl.ds(cur, L)], s16, mask=m)
                    plsc.store_compressed(dstc.at[pl.ds(cur, L)], d16 - lo,
                                          mask=m)
                    if weighted:
                        w16 = w_s[pl.ds(off, L)]
                        plsc.store_compressed(wc.at[pl.ds(cur, L)], w16,
                                              mask=m)
                    return cur + jnp.sum(m.astype(jnp.int32))
                cursor = lax.fori_loop(0, blk // L, cbody, cur0, unroll=4)

                trips = cursor // G
                lax.fori_loop(0, trips, tbody, 0)
                for k in range(G // L):
                    srcc[pl.ds(k * L, L)] = srcc[pl.ds(trips * G + k * L, L)]
                    dstc[pl.ds(k * L, L)] = dstc[pl.ds(trips * G + k * L, L)]
                    if weighted:
                        wc[pl.ds(k * L, L)] = wc[pl.ds(trips * G + k * L, L)]
                return cursor - trips * G
            rem = lax.fori_loop(0, nb, block_body, jnp.int32(0))

            for k in range(G // L):
                srcc[pl.ds(rem + k * L, L)] = jnp.zeros((L,), jnp.int32)
                dstc[pl.ds(rem + k * L, L)] = jnp.full((L,), trash, jnp.int32)
            lax.fori_loop(0, (rem + G - 1) // G, tbody, 0)

            plsc.subcore_barrier()
            for z in range(zpt // ZC):
                roff = sid * zpt + z * ZC
                pltpu.sync_copy(shared.at[pl.ds(roff, ZC)],
                                rows_v.at[pl.ds(0, ZC)])
                pltpu.sync_copy(rows_v.at[pl.ds(0, ZC)],
                                out_hbm.at[pl.ds(rid * PR + roff, ZC)])
            plsc.subcore_barrier()

    scratch = [
        pltpu.VMEM((blk,), jnp.int32),
        pltpu.VMEM((blk,), jnp.int32),
        pltpu.VMEM((blk,), jnp.float32),
        pltpu.VMEM((blk + G,), jnp.int32),
        pltpu.VMEM((blk + G,), jnp.int32),
        pltpu.VMEM((blk + G + L,), jnp.float32),
        pltpu.VMEM((G,), jnp.int32),
        pltpu.VMEM((G,), jnp.int32),
        pltpu.VMEM((G, D), jnp.float32),
        pltpu.SemaphoreType.DMA,
        pltpu.VMEM_SHARED((PR, D), jnp.float32),
    ]
    fn = pl.kernel(
        body,
        out_type=jax.ShapeDtypeStruct((nr * PR, D), jnp.float32),
        mesh=_mesh(),
        scratch_types=scratch,
        compiler_params=pltpu.CompilerParams(needs_layout_passes=False),
    )
    out_pad = fn(x, src_p, dst_p, w_p)
    return out_pad.reshape(nr, PR, D)[:, :R].reshape(nr * R, D)[:n_out]


def _edge_map_call(tab, idx_p, aux_p, blk, fmap):
    E_pad = idx_p.shape[0]
    n_tab = tab.shape[0]
    ap = _cdiv(n_tab + 1, 128) * 128
    tab = _pad1(tab, ap, 0.0)
    nb = E_pad // (NW * blk)
    share = E_pad // NW

    def body(tab_hbm, idx_hbm, aux_hbm, out_hbm, tab_v, idx_s, aux_s, out_s,
             sem):
        cid = lax.axis_index("c")
        sid = lax.axis_index("s")
        wid = cid * NS + sid
        pltpu.sync_copy(tab_hbm, tab_v)
        for bi in range(nb):
            est = wid * share + bi * blk
            pltpu.sync_copy(idx_hbm.at[pl.ds(est, blk)], idx_s)
            if aux_p is not None:
                pltpu.sync_copy(aux_hbm.at[pl.ds(est, blk)], aux_s)

            def gbody(g, _):
                i16 = idx_s[pl.ds(g * L, L)]
                t16 = plsc.load_gather(tab_v, [i16])
                if aux_p is not None:
                    a16 = aux_s[pl.ds(g * L, L)]
                else:
                    a16 = None
                out_s[pl.ds(g * L, L)] = fmap(t16, a16)
                return 0
            lax.fori_loop(0, blk // L, gbody, 0)
            pltpu.sync_copy(out_s, out_hbm.at[pl.ds(est, blk)])

    scratch = [
        pltpu.VMEM((ap,), jnp.float32),
        pltpu.VMEM((blk,), jnp.int32),
        pltpu.VMEM((blk,), jnp.float32),
        pltpu.VMEM((blk,), jnp.float32),
        pltpu.SemaphoreType.DMA,
    ]
    fn = pl.kernel(
        body,
        out_type=jax.ShapeDtypeStruct((E_pad,), jnp.float32),
        mesh=_mesh(),
        scratch_types=scratch,
        compiler_params=pltpu.CompilerParams(needs_layout_passes=False),
    )
    if aux_p is None:
        aux_p = jnp.zeros((E_pad,), jnp.float32)
    return fn(tab, idx_p, aux_p)


def _norm_from_deg(d):
    return jnp.where(d > 0, lax.rsqrt(jnp.maximum(d, 1e-9)), 0.0)


def _tc1(h_game, dpo, dpi, W0):
    n = h_game.shape[0]
    D = h_game.shape[1]
    nbk = n // _BL

    def fn(h_ref, dpo_ref, dpi_ref, w_ref, y_ref, ns_ref, nd_ref):
        dout = jnp.sum(dpo_ref[...], axis=0)
        din = jnp.sum(dpi_ref[...], axis=0)
        ns = _norm_from_deg(dout)
        nd = _norm_from_deg(din)
        ns_ref[...] = ns.reshape(1, 1, _BL)
        nd_ref[...] = nd.reshape(1, 1, _BL)
        y_ref[...] = jnp.dot(h_ref[...] * ns.reshape(_BL, 1), w_ref[...],
                             preferred_element_type=jnp.float32)

    grid = (nbk,)
    return pl.pallas_call(
        fn,
        grid=grid,
        in_specs=[
            pl.BlockSpec((_BL, D), lambda i: (i, 0)),
            pl.BlockSpec((NW, 1, 1, _BL), lambda i: (0, i, 0, 0)),
            pl.BlockSpec((NW, 1, 1, _BL), lambda i: (0, i, 0, 0)),
            pl.BlockSpec((D, D), lambda i: (0, 0)),
        ],
        out_specs=[
            pl.BlockSpec((_BL, D), lambda i: (i, 0)),
            pl.BlockSpec((1, 1, _BL), lambda i: (i, 0, 0)),
            pl.BlockSpec((1, 1, _BL), lambda i: (i, 0, 0)),
        ],
        out_shape=[
            jax.ShapeDtypeStruct((n, D), jnp.float32),
            jax.ShapeDtypeStruct((nbk, 1, _BL), jnp.float32),
            jax.ShapeDtypeStruct((nbk, 1, _BL), jnp.float32),
        ],
    )(h_game, dpo.reshape(NW, nbk, 1, _BL), dpi.reshape(NW, nbk, 1, _BL), W0)


def _tc3(agg1, ns3, nd3, W1):
    n, D = agg1.shape
    nbk = n // _BL

    def fn(a_ref, ns_ref, nd_ref, w_ref, y_ref):
        ns = ns_ref[...].reshape(_BL, 1)
        nd = nd_ref[...].reshape(_BL, 1)
        y_ref[...] = jnp.dot(a_ref[...] * (nd * ns), w_ref[...],
                             preferred_element_type=jnp.float32)

    return pl.pallas_call(
        fn,
        grid=(nbk,),
        in_specs=[
            pl.BlockSpec((_BL, D), lambda i: (i, 0)),
            pl.BlockSpec((1, 1, _BL), lambda i: (i, 0, 0)),
            pl.BlockSpec((1, 1, _BL), lambda i: (i, 0, 0)),
            pl.BlockSpec((D, D), lambda i: (0, 0)),
        ],
        out_specs=pl.BlockSpec((_BL, D), lambda i: (i, 0)),
        out_shape=jax.ShapeDtypeStruct((n, D), jnp.float32),
    )(agg1, ns3, nd3, W1)


def _tc5(agg2, nd3):
    n, D = agg2.shape
    nbk = n // _BL

    def fn(a_ref, nd_ref, y_ref):
        y_ref[...] = a_ref[...] * nd_ref[...].reshape(_BL, 1)

    return pl.pallas_call(
        fn,
        grid=(nbk,),
        in_specs=[
            pl.BlockSpec((_BL, D), lambda i: (i, 0)),
            pl.BlockSpec((1, 1, _BL), lambda i: (i, 0, 0)),
        ],
        out_specs=pl.BlockSpec((_BL, D), lambda i: (i, 0)),
        out_shape=jax.ShapeDtypeStruct((n, D), jnp.float32),
    )(agg2, nd3)


def _tc7(ue, sgu, cgu, cso, Wsi, Wni, bi, Wg, al, ar):
    n, D = ue.shape
    nbk = n // _BL

    def fn(ue_ref, sgu_ref, cgu_ref, cso_ref, wsi_ref, wni_ref, bi_ref,
           wg_ref, al_ref, ar_ref, hua_ref, el_ref, er_ref, ivs_ref):
        cg = jnp.sum(cgu_ref[...], axis=0)
        cs = jnp.sum(cso_ref[...], axis=0)
        ivg = 1.0 / jnp.maximum(cg, 1.0)
        ivs = 1.0 / jnp.maximum(cs, 1.0)
        ivs_ref[...] = ivs.reshape(1, 1, _BL)
        hua = (jnp.dot(ue_ref[...], wsi_ref[...],
                       preferred_element_type=jnp.float32)
               + jnp.dot(sgu_ref[...], wni_ref[...],
                         preferred_element_type=jnp.float32)
               * ivg.reshape(_BL, 1)
               + bi_ref[...])
        hua_ref[...] = hua
        feat = jnp.dot(hua, wg_ref[...], preferred_element_type=jnp.float32)
        el = jnp.sum(feat * al_ref[...], axis=1)
        er = jnp.sum(feat * ar_ref[...], axis=1)
        el_ref[...] = el.reshape(1, 1, _BL)
        er_ref[...] = er.reshape(1, 1, _BL)

    return pl.pallas_call(
        fn,
        grid=(nbk,),
        in_specs=[
            pl.BlockSpec((_BL, D), lambda i: (i, 0)),
            pl.BlockSpec((_BL, D), lambda i: (i, 0)),
            pl.BlockSpec((NW, 1, 1, _BL), lambda i: (0, i, 0, 0)),
            pl.BlockSpec((NW, 1, 1, _BL), lambda i: (0, i, 0, 0)),
            pl.BlockSpec((D, D), lambda i: (0, 0)),
            pl.BlockSpec((D, D), lambda i: (0, 0)),
            pl.BlockSpec((1, D), lambda i: (0, 0)),
            pl.BlockSpec((D, D), lambda i: (0, 0)),
            pl.BlockSpec((1, D), lambda i: (0, 0)),
            pl.BlockSpec((1, D), lambda i: (0, 0)),
        ],
        out_specs=[
            pl.BlockSpec((_BL, D), lambda i: (i, 0)),
            pl.BlockSpec((1, 1, _BL), lambda i: (i, 0, 0)),
            pl.BlockSpec((1, 1, _BL), lambda i: (i, 0, 0)),
            pl.BlockSpec((1, 1, _BL), lambda i: (i, 0, 0)),
        ],
        out_shape=[
            jax.ShapeDtypeStruct((n, D), jnp.float32),
            jax.ShapeDtypeStruct((nbk, 1, _BL), jnp.float32),
            jax.ShapeDtypeStruct((nbk, 1, _BL), jnp.float32),
            jax.ShapeDtypeStruct((nbk, 1, _BL), jnp.float32),
        ],
    )(ue, sgu, cgu.reshape(NW, nbk, 1, _BL), cso.reshape(NW, nbk, 1, _BL),
      Wsi, Wni, bi.reshape(1, D), Wg, al.reshape(1, D), ar.reshape(1, D))


def _tc11(esp):
    nw, n = esp.shape
    nbk = n // _BL

    def fn(p_ref, o_ref):
        o_ref[...] = jnp.sum(p_ref[...], axis=0).reshape(1, 1, _BL)

    return pl.pallas_call(
        fn,
        grid=(nbk,),
        in_specs=[pl.BlockSpec((NW, 1, 1, _BL), lambda i: (0, i, 0, 0))],
        out_specs=pl.BlockSpec((1, 1, _BL), lambda i: (i, 0, 0)),
        out_shape=jax.ShapeDtypeStruct((nbk, 1, _BL), jnp.float32),
    )(esp.reshape(NW, nbk, 1, _BL))


def _tc14(ue, hua, ssoc, ivs3, Wss, Wns, bs, wu, wa, ws):
    n, D = ue.shape
    nbk = n // _BL

    def fn(ue_ref, hua_ref, ss_ref, ivs_ref, wss_ref, wns_ref, bs_ref, o_ref):
        ivs = ivs_ref[...].reshape(_BL, 1)
        hus = (jnp.dot(ue_ref[...], wss_ref[...],
                       preferred_element_type=jnp.float32)
               + jnp.dot(ss_ref[...], wns_ref[...],
                         preferred_element_type=jnp.float32) * ivs
               + bs_ref[...])
        o_ref[...] = wu * ue_ref[...] + wa * hua_ref[...] + ws * hus

    return pl.pallas_call(
        fn,
        grid=(nbk,),
        in_specs=[
            pl.BlockSpec((_BL, D), lambda i: (i, 0)),
            pl.BlockSpec((_BL, D), lambda i: (i, 0)),
            pl.BlockSpec((_BL, D), lambda i: (i, 0)),
            pl.BlockSpec((1, 1, _BL), lambda i: (i, 0, 0)),
            pl.BlockSpec((D, D), lambda i: (0, 0)),
            pl.BlockSpec((D, D), lambda i: (0, 0)),
            pl.BlockSpec((1, D), lambda i: (0, 0)),
        ],
        out_specs=pl.BlockSpec((_BL, D), lambda i: (i, 0)),
        out_shape=jax.ShapeDtypeStruct((n, D), jnp.float32),
    )(ue, hua, ssoc, ivs3, Wss, Wns, bs.reshape(1, D))


def kernel(h_game, gu_weight, user_embedding, item_embedding, W_gc0, W_gc1,
           W_self_i, W_neigh_i, b_i, W_gat, attn_l, attn_r, W_self_s,
           W_neigh_s, b_s, item_edge_index, gu_src, gu_dst,
           social_edge_index):
    ni = h_game.shape[0]
    nu = user_embedding.shape[0]
    ei = item_edge_index.shape[1]
    eg = gu_src.shape[0]
    es = social_edge_index.shape[1]

    iblk, ei_pad = _pad_edges(ei)
    gblk, eg_pad = _pad_edges(eg)
    sblk, es_pad = _pad_edges(es)

    isrc = _pad1(item_edge_index[0], ei_pad, 0)
    idst = _pad1(item_edge_index[1], ei_pad, ni)
    gus = _pad1(gu_src, eg_pad, 0)
    gud = _pad1(gu_dst, eg_pad, nu)
    guw = _pad1(gu_weight, eg_pad, 0.0)
    ssrc = _pad1(social_edge_index[0], es_pad, 0)
    sdst = _pad1(social_edge_index[1], es_pad, nu)

    zi = jnp.zeros((ei_pad,), jnp.float32)
    zg = jnp.zeros((eg_pad,), jnp.float32)
    zs = jnp.zeros((es_pad,), jnp.float32)

    dpo = _hist_call(isrc, zi, ni, iblk, False)
    dpi = _hist_call(idst, zi, ni, iblk, False)
    cgu = _hist_call(gud, zg, nu, gblk, False)
    cso = _hist_call(sdst, zs, nu, sblk, False)

    y0, ns3, nd3 = _tc1(h_game, dpo, dpi, W_gc0)
    agg1 = _rowscatter_call(y0, isrc, idst, zi, ni, iblk, False)
    y1 = _tc3(agg1, ns3, nd3, W_gc1)
    agg2 = _rowscatter_call(y1, isrc, idst, zi, ni, iblk, False)
    h2 = _tc5(agg2, nd3)

    sgu = _rowscatter_call(h2, gus, gud, guw, nu, gblk, True)
    hua, el3, er3, ivs3 = _tc7(user_embedding, sgu, cgu, cso, W_self_i,
                               W_neigh_i, b_i, W_gat, attn_l, attn_r)

    el = el3.reshape(nu)
    er = er3.reshape(nu)
    tmp = _edge_map_call(el, ssrc, None, sblk, lambda t, a: t)

    def _exp_leaky(t, a):
        e = a + t
        e = jnp.where(e >= 0, e, 0.2 * e)
        return jnp.exp(e)
    ex = _edge_map_call(er, sdst, tmp, sblk, _exp_leaky)
    esp = _hist_call(sdst, ex, nu, sblk, True)
    esum3 = _tc11(esp)
    alpha = _edge_map_call(esum3.reshape(nu), sdst, ex, sblk,
                           lambda t, a: a / (t + 1e-9))

    ssoc = _rowscatter_call(user_embedding, ssrc, sdst, alpha, nu, sblk, True)

    wu = 1.0 - 0.1 - 0.2
    user_out = _tc14(user_embedding, hua, ssoc, ivs3, W_self_s, W_neigh_s,
                     b_s, wu, 0.2, 0.1)
    return user_out, item_embedding
